# baseline (speedup 1.0000x reference)
import functools

import jax
import jax.numpy as jnp
from jax import lax
from jax.experimental import pallas as pl
from jax.experimental.pallas import tpu as pltpu
from jax.experimental.pallas import tpu_sc as plsc

M = 1024
INPUT_DIM = 256
HIDDEN = 64
B = 4
N = 196
T = B * N
TP = 1024

NC = 1
NS = 16
NW = NC * NS
BPW = TP // NW

DPAD = 128


def _bf16x1_dot(a, b):
    return lax.dot_general(a.astype(jnp.bfloat16), b.astype(jnp.bfloat16),
                           (((1,), (1,)), ((), ())),
                           preferred_element_type=jnp.float32)


def _tc_score_kernel(x_ref, m1_ref, w1_ref, b1_ref, w2_ref, b2_ref,
                     idx_ref, outa_ref, gt_ref):
    W1b = w1_ref[...].astype(jnp.bfloat16)
    embs = [
        lax.dot_general(x_ref[b].astype(jnp.bfloat16), W1b,
                        (((0,), (1,)), ((), ())),
                        preferred_element_type=jnp.float32)
        for b in range(B)
    ]
    emb = jnp.concatenate(embs, axis=0)
    emb = jnp.maximum(emb + b1_ref[...], 0.0)

    mx = jnp.max(emb, axis=1, keepdims=True)
    ex = jnp.exp(emb - mx)
    p = ex / jnp.sum(ex, axis=1, keepdims=True)
    logp = jnp.log(p)

    m1 = m1_ref[...]
    cmx = jnp.max(m1, axis=1, keepdims=True)
    ce = jnp.exp(m1 - cmx)
    m = ce / jnp.sum(ce, axis=1, keepdims=True)
    ent = jnp.sum(m * jnp.log(m), axis=1, keepdims=True)

    crossT = _bf16x1_dot(m, logp)
    klT = ent - crossT

    mn = jnp.min(klT, axis=0, keepdims=True)
    ii = lax.broadcasted_iota(jnp.int32, (M, T), 0)
    idx_ref[:, 0:T] = jnp.min(jnp.where(klT == mn, ii, jnp.int32(2**30)),
                              axis=0, keepdims=True)
    idx_ref[:, T:TP] = jnp.zeros((1, TP - T), jnp.int32)

    w2a = w2_ref[:, 0:HIDDEN]
    w2b = w2_ref[:, HIDDEN:2 * HIDDEN]
    outa_ref[0:T, :] = _bf16x1_dot(emb, w2a) + b2_ref[...]

    gt = _bf16x1_dot(m1, w2b)
    gt_ref[...] = jnp.concatenate(
        [gt, jnp.zeros((M, DPAD - HIDDEN), jnp.float32)], axis=1)


@functools.lru_cache(maxsize=1)
def _make_sc_out():
    mesh = plsc.VectorSubcoreMesh(core_axis_name="c", subcore_axis_name="s", num_cores=1)

    @functools.partial(
        pl.kernel,
        mesh=mesh,
        out_type=jax.ShapeDtypeStruct((TP, HIDDEN), jnp.float32),
        scratch_types=[
            pltpu.VMEM((BPW,), jnp.int32),
            pltpu.VMEM((BPW, DPAD), jnp.float32),
            pltpu.VMEM((BPW, HIDDEN), jnp.float32),
            pltpu.SemaphoreType.DMA,
            pltpu.SemaphoreType.DMA,
        ],
    )
    def _sc_out(gt_hbm, idx_hbm, outa_hbm, out_hbm, idx_v, rows_v, acc_v,
                sem, sem2):
        wid = lax.axis_index("s") * NC + lax.axis_index("c")
        base = wid * BPW
        with jax.named_scope("sc_idx_outa"):
            outa_cp = pltpu.async_copy(outa_hbm.at[pl.ds(base, BPW)], acc_v, sem2)
            pltpu.sync_copy(idx_hbm.at[pl.ds(base, BPW)], idx_v)
        NSPLIT = 4
        CH = BPW // NSPLIT
        with jax.named_scope("sc_gather"):
            gathers = [
                pltpu.async_copy(gt_hbm.at[idx_v.at[pl.ds(k * CH, CH)]],
                                 rows_v.at[pl.ds(k * CH, CH)], sem)
                for k in range(NSPLIT)
            ]
            outa_cp.wait()
            for g in gathers:
                g.wait()

        def body(i, carry):
            for c in range(HIDDEN // 16):
                sl = (i, pl.ds(c * 16, 16))
                acc_v[sl] = acc_v[sl] + rows_v[sl]
            return carry

        with jax.named_scope("sc_adds"):
            lax.fori_loop(0, BPW, body, 0)
        with jax.named_scope("sc_store"):
            pltpu.sync_copy(acc_v, out_hbm.at[pl.ds(base, BPW)])

    return _sc_out


def kernel(input, m1, W1, b1, W2, b2):
    x = jnp.squeeze(input, axis=-1)

    idx, outa, gt = pl.pallas_call(
        _tc_score_kernel,
        out_shape=(
            jax.ShapeDtypeStruct((1, TP), jnp.int32),
            jax.ShapeDtypeStruct((TP, HIDDEN), jnp.float32),
            jax.ShapeDtypeStruct((M, DPAD), jnp.float32),
        ),
    )(x, m1, W1, b1.reshape(1, HIDDEN), W2, b2.reshape(1, HIDDEN))

    out_t = _make_sc_out()(gt, idx.reshape(TP), outa)

    out = jnp.transpose(out_t[:T].reshape(B, N, HIDDEN), (0, 2, 1))
    return out[..., None]


# speedup vs baseline: 1.0011x; 1.0011x over previous
import functools

import jax
import jax.numpy as jnp
from jax import lax
from jax.experimental import pallas as pl
from jax.experimental.pallas import tpu as pltpu
from jax.experimental.pallas import tpu_sc as plsc

M = 1024
INPUT_DIM = 256
HIDDEN = 64
B = 4
N = 196
T = B * N
TP = 1024

NC = 1
NS = 16
NW = NC * NS
BPW = TP // NW

DPAD = 128


def _bf16x1_dot(a, b):
    return lax.dot_general(a.astype(jnp.bfloat16), b.astype(jnp.bfloat16),
                           (((1,), (1,)), ((), ())),
                           preferred_element_type=jnp.float32)


def _tc_score_kernel(x_ref, m1_ref, w1_ref, b1_ref, w2_ref, b2_ref,
                     idx_ref, outa_ref, gt_ref):
    W1b = w1_ref[...].astype(jnp.bfloat16)
    embs = [
        lax.dot_general(x_ref[b].astype(jnp.bfloat16), W1b,
                        (((0,), (1,)), ((), ())),
                        preferred_element_type=jnp.float32)
        for b in range(B)
    ]
    emb = jnp.concatenate(embs, axis=0)
    emb = jnp.maximum(emb + b1_ref[...], 0.0)

    mx = jnp.max(emb, axis=1, keepdims=True)
    ex = jnp.exp(emb - mx)
    p = ex / jnp.sum(ex, axis=1, keepdims=True)
    logp = jnp.log(p)

    m1 = m1_ref[...]
    cmx = jnp.max(m1, axis=1, keepdims=True)
    ce = jnp.exp(m1 - cmx)
    m = ce / jnp.sum(ce, axis=1, keepdims=True)
    ent = jnp.sum(m * jnp.log(m), axis=1, keepdims=True)

    crossT = _bf16x1_dot(m, logp)
    klT = ent - crossT

    mn = jnp.min(klT, axis=0, keepdims=True)
    ii = lax.broadcasted_iota(jnp.int32, (M, T), 0)
    idx_ref[:, 0:T] = jnp.min(jnp.where(klT == mn, ii, jnp.int32(2**30)),
                              axis=0, keepdims=True)
    idx_ref[:, T:TP] = jnp.zeros((1, TP - T), jnp.int32)

    w2a = w2_ref[:, 0:HIDDEN]
    w2b = w2_ref[:, HIDDEN:2 * HIDDEN]
    outa_ref[0:T, :] = _bf16x1_dot(emb, w2a) + b2_ref[...]

    gt = _bf16x1_dot(m1, w2b)
    gt_ref[...] = jnp.concatenate(
        [gt, jnp.zeros((M, DPAD - HIDDEN), jnp.float32)], axis=1)


@functools.lru_cache(maxsize=1)
def _make_sc_out():
    mesh = plsc.VectorSubcoreMesh(core_axis_name="c", subcore_axis_name="s", num_cores=1)

    @functools.partial(
        pl.kernel,
        mesh=mesh,
        out_type=jax.ShapeDtypeStruct((TP, HIDDEN), jnp.float32),
        scratch_types=[
            pltpu.VMEM((BPW,), jnp.int32),
            pltpu.VMEM((BPW, DPAD), jnp.float32),
            pltpu.VMEM((BPW, HIDDEN), jnp.float32),
            pltpu.SemaphoreType.DMA,
            pltpu.SemaphoreType.DMA,
        ],
    )
    def _sc_out(gt_hbm, idx_hbm, outa_hbm, out_hbm, idx_v, rows_v, acc_v,
                sem, sem2):
        wid = lax.axis_index("s") * NC + lax.axis_index("c")
        base = wid * BPW
        outa_cp = pltpu.async_copy(outa_hbm.at[pl.ds(base, BPW)], acc_v, sem2)
        pltpu.sync_copy(idx_hbm.at[pl.ds(base, BPW)], idx_v)
        NSPLIT = 2
        CH = BPW // NSPLIT
        gathers = [
            pltpu.async_copy(gt_hbm.at[idx_v.at[pl.ds(k * CH, CH)]],
                             rows_v.at[pl.ds(k * CH, CH)], sem)
            for k in range(NSPLIT)
        ]
        outa_cp.wait()
        for g in gathers:
            g.wait()

        def body(i, carry):
            for c in range(HIDDEN // 16):
                sl = (i, pl.ds(c * 16, 16))
                acc_v[sl] = acc_v[sl] + rows_v[sl]
            return carry

        lax.fori_loop(0, BPW, body, 0)
        pltpu.sync_copy(acc_v, out_hbm.at[pl.ds(base, BPW)])

    return _sc_out


def kernel(input, m1, W1, b1, W2, b2):
    x = jnp.squeeze(input, axis=-1)

    idx, outa, gt = pl.pallas_call(
        _tc_score_kernel,
        out_shape=(
            jax.ShapeDtypeStruct((1, TP), jnp.int32),
            jax.ShapeDtypeStruct((TP, HIDDEN), jnp.float32),
            jax.ShapeDtypeStruct((M, DPAD), jnp.float32),
        ),
    )(x, m1, W1, b1.reshape(1, HIDDEN), W2, b2.reshape(1, HIDDEN))

    out_t = _make_sc_out()(gt, idx.reshape(TP), outa)

    out = jnp.transpose(out_t[:T].reshape(B, N, HIDDEN), (0, 2, 1))
    return out[..., None]
